# baseline (speedup 1.0000x reference)
import functools

import jax
import jax.numpy as jnp
from jax import lax
from jax.experimental import pallas as pl
from jax.experimental.pallas import tpu as pltpu
from jax.experimental.pallas import tpu_sc as plsc

D = 256
SCALE = 4
C = D // SCALE
V = C * C * C
ROW = 16
CHUNK = 2048
LANES = 16

NC = 2
NS = 16
NW = NC * NS


def _sc_body(n_chunks, xs0, xs1, xs2, cells, out,
             x0_v, x1_v, x2_v, f0_v, f1_v, f2_v, idx_v, rows_v, tr_v, out_v,
             sem):
    wid = lax.axis_index("s") * NC + lax.axis_index("c")
    base_w = wid * (n_chunks * CHUNK)
    iota = lax.iota(jnp.int32, LANES)
    tr_pat = iota * CHUNK

    def chunk_body(ch, _):
        base = base_w + ch * CHUNK
        pltpu.sync_copy(xs0.at[pl.ds(base, CHUNK)], x0_v)
        pltpu.sync_copy(xs1.at[pl.ds(base, CHUNK)], x1_v)
        pltpu.sync_copy(xs2.at[pl.ds(base, CHUNK)], x2_v)

        def idx_body(i, _):
            sl = pl.ds(i * LANES, LANES)
            s0 = x0_v[sl] * jnp.float32(D)
            s1 = x1_v[sl] * jnp.float32(D)
            s2 = x2_v[sl] * jnp.float32(D)
            i0 = lax.bitwise_and(s0.astype(jnp.int32), -SCALE)
            i1 = lax.bitwise_and(s1.astype(jnp.int32), -SCALE)
            i2 = lax.bitwise_and(s2.astype(jnp.int32), -SCALE)
            f0_v[sl] = (s0 - i0.astype(jnp.float32)) * jnp.float32(1.0 / SCALE)
            f1_v[sl] = (s1 - i1.astype(jnp.float32)) * jnp.float32(1.0 / SCALE)
            f2_v[sl] = (s2 - i2.astype(jnp.float32)) * jnp.float32(1.0 / SCALE)
            c0 = lax.shift_right_arithmetic(i0, 2)
            c1 = lax.shift_right_arithmetic(i1, 2)
            c2 = lax.shift_right_arithmetic(i2, 2)
            idx_v[sl] = (c0 * C + c1) * C + c2
            return 0

        lax.fori_loop(0, CHUNK // LANES, idx_body, 0)

        copies = []
        for j in range(CHUNK // 128):
            isl = pl.ds(j * 128, 128)
            copies.append(
                pltpu.async_copy(cells.at[idx_v.at[isl]],
                                 rows_v.at[isl, :], sem))
        for cp in copies:
            cp.wait()

        def tr_body(t, _):
            for k in range(8):
                p = t * 8 + k
                plsc.store_scatter(tr_v, [tr_pat + p], rows_v[p, :])
            return 0

        lax.fori_loop(0, CHUNK // 8, tr_body, 0)

        def comp_body(i, _):
            sl = pl.ds(i * LANES, LANES)
            f0 = f0_v[sl]
            f1 = f1_v[sl]
            f2 = f2_v[sl]
            g0 = jnp.float32(1.0) - f0
            g1 = jnp.float32(1.0) - f1
            g2 = jnp.float32(1.0) - f2
            q00 = g1 * g2
            q10 = f1 * g2
            q01 = g1 * f2
            q11 = f1 * f2
            def corner(j):
                return tr_v[pl.ds(j * CHUNK + i * LANES, LANES)]

            acc = corner(0) * (g0 * q00)
            acc = acc + corner(1) * (f0 * q00)
            acc = acc + corner(2) * (g0 * q10)
            acc = acc + corner(3) * (f0 * q10)
            acc = acc + corner(4) * (g0 * q01)
            acc = acc + corner(5) * (f0 * q01)
            acc = acc + corner(6) * (g0 * q11)
            acc = acc + corner(7) * (f0 * q11)
            out_v[sl] = acc
            return 0

        lax.fori_loop(0, CHUNK // LANES, comp_body, 0)
        pltpu.sync_copy(out_v, out.at[pl.ds(base, CHUNK)])
        return 0

    lax.fori_loop(0, n_chunks, chunk_body, 0)


def kernel(xs, data):
    n = xs.shape[0]
    n_chunks = n // (NW * CHUNK)

    d3 = data[:, :, :, 0]
    a = jnp.concatenate([d3[::SCALE], d3[D - 1:]], 0)
    b = jnp.concatenate([a[:, ::SCALE], a[:, D - 1:]], 1)
    c = jnp.concatenate([b[:, :, ::SCALE], b[:, :, D - 1:]], 2)
    corners = [c[dz:dz + C, dy:dy + C, dx:dx + C]
               for dz in (0, 1) for dy in (0, 1) for dx in (0, 1)]
    cells = jnp.stack(corners + corners[:8], axis=-1).reshape(V, ROW)

    xs0 = xs[:, 0]
    xs1 = xs[:, 1]
    xs2 = xs[:, 2]

    mesh = plsc.VectorSubcoreMesh(core_axis_name="c", subcore_axis_name="s")
    run = functools.partial(
        pl.kernel,
        mesh=mesh,
        compiler_params=pltpu.CompilerParams(
            needs_layout_passes=False, use_tc_tiling_on_sc=False),
        out_type=jax.ShapeDtypeStruct((n,), jnp.float32),
        scratch_types=[
            pltpu.VMEM((CHUNK,), jnp.float32),
            pltpu.VMEM((CHUNK,), jnp.float32),
            pltpu.VMEM((CHUNK,), jnp.float32),
            pltpu.VMEM((CHUNK,), jnp.float32),
            pltpu.VMEM((CHUNK,), jnp.float32),
            pltpu.VMEM((CHUNK,), jnp.float32),
            pltpu.VMEM((CHUNK,), jnp.int32),
            pltpu.VMEM((CHUNK, ROW), jnp.float32),
            pltpu.VMEM((CHUNK * ROW,), jnp.float32),
            pltpu.VMEM((CHUNK,), jnp.float32),
            pltpu.SemaphoreType.DMA,
        ],
    )(functools.partial(_sc_body, n_chunks))
    out = run(xs0, xs1, xs2, cells)
    return out.reshape(n, 1)


# speedup vs baseline: 1.5381x; 1.5381x over previous
import functools

import jax
import jax.numpy as jnp
from jax import lax
from jax.experimental import pallas as pl
from jax.experimental.pallas import tpu as pltpu
from jax.experimental.pallas import tpu_sc as plsc

D = 256
SCALE = 4
C = D // SCALE
V = C * C * C
ROW = 16
CHUNK = 2048
LANES = 16

NC = 2
NS = 16
NW = NC * NS


def _sc_body(n_chunks, xs0, xs1, xs2, cells, out,
             x0_v, x1_v, x2_v, f0_v, f1_v, f2_v, idx_v, rows_v, out_v,
             sem):
    wid = lax.axis_index("s") * NC + lax.axis_index("c")
    base_w = wid * (n_chunks * CHUNK)
    iota = lax.iota(jnp.int32, LANES)

    def chunk_body(ch, _):
        base = base_w + ch * CHUNK
        pltpu.sync_copy(xs0.at[pl.ds(base, CHUNK)], x0_v)
        pltpu.sync_copy(xs1.at[pl.ds(base, CHUNK)], x1_v)
        pltpu.sync_copy(xs2.at[pl.ds(base, CHUNK)], x2_v)

        def idx_body(i, _):
            sl = pl.ds(i * LANES, LANES)
            s0 = x0_v[sl] * jnp.float32(D)
            s1 = x1_v[sl] * jnp.float32(D)
            s2 = x2_v[sl] * jnp.float32(D)
            i0 = lax.bitwise_and(s0.astype(jnp.int32), -SCALE)
            i1 = lax.bitwise_and(s1.astype(jnp.int32), -SCALE)
            i2 = lax.bitwise_and(s2.astype(jnp.int32), -SCALE)
            f0_v[sl] = (s0 - i0.astype(jnp.float32)) * jnp.float32(1.0 / SCALE)
            f1_v[sl] = (s1 - i1.astype(jnp.float32)) * jnp.float32(1.0 / SCALE)
            f2_v[sl] = (s2 - i2.astype(jnp.float32)) * jnp.float32(1.0 / SCALE)
            c0 = lax.shift_right_arithmetic(i0, 2)
            c1 = lax.shift_right_arithmetic(i1, 2)
            c2 = lax.shift_right_arithmetic(i2, 2)
            idx_v[sl] = (c0 * C + c1) * C + c2
            return 0

        lax.fori_loop(0, CHUNK // LANES, idx_body, 0)

        copies = []
        for j in range(CHUNK // 128):
            isl = pl.ds(j * 128, 128)
            copies.append(
                pltpu.async_copy(cells.at[idx_v.at[isl]],
                                 rows_v.at[isl, :], sem))
        for cp in copies:
            cp.wait()


        def comp_body(i, _):
            sl = pl.ds(i * LANES, LANES)
            f0 = f0_v[sl]
            f1 = f1_v[sl]
            f2 = f2_v[sl]
            g0 = jnp.float32(1.0) - f0
            g1 = jnp.float32(1.0) - f1
            g2 = jnp.float32(1.0) - f2
            q00 = g1 * g2
            q10 = f1 * g2
            q01 = g1 * f2
            q11 = f1 * f2
            row = i * LANES + iota

            def corner(j):
                col = jnp.full((LANES,), j, jnp.int32)
                return plsc.load_gather(rows_v, [row, col])

            acc = corner(0) * (g0 * q00)
            acc = acc + corner(1) * (f0 * q00)
            acc = acc + corner(2) * (g0 * q10)
            acc = acc + corner(3) * (f0 * q10)
            acc = acc + corner(4) * (g0 * q01)
            acc = acc + corner(5) * (f0 * q01)
            acc = acc + corner(6) * (g0 * q11)
            acc = acc + corner(7) * (f0 * q11)
            out_v[sl] = acc
            return 0

        lax.fori_loop(0, CHUNK // LANES, comp_body, 0)
        pltpu.sync_copy(out_v, out.at[pl.ds(base, CHUNK)])
        return 0

    lax.fori_loop(0, n_chunks, chunk_body, 0)


def kernel(xs, data):
    n = xs.shape[0]
    n_chunks = n // (NW * CHUNK)

    d3 = data[:, :, :, 0]
    a = jnp.concatenate([d3[::SCALE], d3[D - 1:]], 0)
    b = jnp.concatenate([a[:, ::SCALE], a[:, D - 1:]], 1)
    c = jnp.concatenate([b[:, :, ::SCALE], b[:, :, D - 1:]], 2)
    corners = [c[dz:dz + C, dy:dy + C, dx:dx + C]
               for dz in (0, 1) for dy in (0, 1) for dx in (0, 1)]
    cells = jnp.stack(corners + corners[:8], axis=-1).reshape(V, ROW)

    xs0 = xs[:, 0]
    xs1 = xs[:, 1]
    xs2 = xs[:, 2]

    mesh = plsc.VectorSubcoreMesh(core_axis_name="c", subcore_axis_name="s")
    run = functools.partial(
        pl.kernel,
        mesh=mesh,
        compiler_params=pltpu.CompilerParams(
            needs_layout_passes=False, use_tc_tiling_on_sc=False),
        out_type=jax.ShapeDtypeStruct((n,), jnp.float32),
        scratch_types=[
            pltpu.VMEM((CHUNK,), jnp.float32),
            pltpu.VMEM((CHUNK,), jnp.float32),
            pltpu.VMEM((CHUNK,), jnp.float32),
            pltpu.VMEM((CHUNK,), jnp.float32),
            pltpu.VMEM((CHUNK,), jnp.float32),
            pltpu.VMEM((CHUNK,), jnp.float32),
            pltpu.VMEM((CHUNK,), jnp.int32),
            pltpu.VMEM((CHUNK, ROW), jnp.float32),
            pltpu.VMEM((CHUNK,), jnp.float32),
            pltpu.SemaphoreType.DMA,
        ],
    )(functools.partial(_sc_body, n_chunks))
    out = run(xs0, xs1, xs2, cells)
    return out.reshape(n, 1)
